# baseline (speedup 1.0000x reference)
import functools

import jax
import jax.numpy as jnp
from jax import lax
from jax.experimental import pallas as pl
from jax.experimental.pallas import tpu as pltpu
from jax.experimental.pallas import tpu_sc as plsc

_NC = 2
_NS = 16
_NW = _NC * _NS
_CHUNK = 1024


@functools.lru_cache(maxsize=None)
def _build_gather(batch: int):
  assert batch % (8 * _NW) == 0
  b_per_w = batch // _NW
  chunk = min(_CHUNK, b_per_w)
  n_chunks = b_per_w // chunk
  assert b_per_w % chunk == 0

  mesh = plsc.VectorSubcoreMesh(
      core_axis_name="c", subcore_axis_name="s", num_cores=_NC, num_subcores=_NS)

  @functools.partial(
      pl.kernel,
      out_type=jax.ShapeDtypeStruct((batch,), jnp.float32),
      mesh=mesh,
      scratch_types=[
          pltpu.VMEM((b_per_w,), jnp.int32),
          pltpu.VMEM((b_per_w,), jnp.float32),
          pltpu.SemaphoreType.DMA,
      ],
  )
  def gather_kernel(table_hbm, idx_hbm, out_hbm, idx_v, vals_v, sem):
    wid = lax.axis_index("s") * _NC + lax.axis_index("c")
    base = wid * b_per_w
    table = table_hbm.at[0]
    pltpu.sync_copy(idx_hbm.at[pl.ds(base, b_per_w)], idx_v)
    descs = [
        pltpu.async_copy(
            table.at[idx_v.at[pl.ds(j * chunk, chunk)]],
            vals_v.at[pl.ds(j * chunk, chunk)],
            sem,
        )
        for j in range(n_chunks)
    ]
    for d in descs:
      d.wait()
    pltpu.sync_copy(vals_v, out_hbm.at[pl.ds(base, b_per_w)])

  return gather_kernel


def kernel(state, W):
  idx = state.astype(jnp.int32)
  vals = _build_gather(idx.shape[0])(W, idx)
  return vals[:, None]


# speedup vs baseline: 1.0469x; 1.0469x over previous
import functools

import jax
import jax.numpy as jnp
from jax import lax
from jax.experimental import pallas as pl
from jax.experimental.pallas import tpu as pltpu
from jax.experimental.pallas import tpu_sc as plsc

_NC = 1
_NS = 16
_NW = _NC * _NS
_CHUNK = 512


@functools.lru_cache(maxsize=None)
def _build_gather(batch: int):
  assert batch % (8 * _NW) == 0
  b_per_w = batch // _NW
  chunk = min(_CHUNK, b_per_w)
  n_chunks = b_per_w // chunk
  assert b_per_w % chunk == 0

  mesh = plsc.VectorSubcoreMesh(
      core_axis_name="c", subcore_axis_name="s", num_cores=_NC, num_subcores=_NS)

  @functools.partial(
      pl.kernel,
      out_type=jax.ShapeDtypeStruct((batch,), jnp.float32),
      mesh=mesh,
      scratch_types=[
          pltpu.VMEM((b_per_w,), jnp.int32),
          pltpu.VMEM((b_per_w,), jnp.float32),
          pltpu.SemaphoreType.DMA,
          pltpu.SemaphoreType.DMA,
      ],
  )
  def gather_kernel(table_hbm, idx_hbm, out_hbm, idx_v, vals_v, gsem, osem):
    wid = lax.axis_index("s") * _NC + lax.axis_index("c")
    base = wid * b_per_w
    table = table_hbm.at[0]
    gathers = []
    for j in range(n_chunks):
      sl = pl.ds(j * chunk, chunk)
      pltpu.sync_copy(idx_hbm.at[pl.ds(base + j * chunk, chunk)], idx_v.at[sl])
      gathers.append(
          pltpu.async_copy(table.at[idx_v.at[sl]], vals_v.at[sl], gsem))
    outs = []
    for j in range(n_chunks):
      sl = pl.ds(j * chunk, chunk)
      gathers[j].wait()
      outs.append(
          pltpu.async_copy(
              vals_v.at[sl], out_hbm.at[pl.ds(base + j * chunk, chunk)], osem))
    for d in outs:
      d.wait()

  return gather_kernel


def kernel(state, W):
  idx = state.astype(jnp.int32)
  vals = _build_gather(idx.shape[0])(W, idx)
  return vals[:, None]
